# baseline (speedup 1.0000x reference)
import functools

import jax
import jax.numpy as jnp
from jax import lax
from jax.experimental import pallas as pl
from jax.experimental.pallas import tpu as pltpu
from jax.experimental.pallas import tpu_sc as plsc

C = 128
N = 10000
E = 320000
NC = 2
NS = 16
NW = NC * NS
EPW = E // NW
CHUNK = 112
NCHUNK = 90
EPW_PAD = NCHUNK * CHUNK
NPAD = 10240
RPS = NPAD // NS
LANE = 16
NVEC = C // LANE

ROW_BLK = 2000


def _proj_body(x_ref, wd_ref, wr_ref, bd_ref, pd_ref, ps_ref):
    xb = x_ref[...]
    pd_ref[...] = (
        jnp.dot(xb, wd_ref[...], preferred_element_type=jnp.float32) + bd_ref[...]
    )
    ps_ref[...] = jnp.dot(xb, wr_ref[...], preferred_element_type=jnp.float32)


def _combine_proj_body(x_ref, agg_ref, wd_ref, wr_ref, bd_ref,
                       h_ref, pd_ref, ps_ref):
    h = jnp.maximum(x_ref[...] + agg_ref[0] + agg_ref[1], 0.0)
    h_ref[...] = h
    pd_ref[...] = (
        jnp.dot(h, wd_ref[...], preferred_element_type=jnp.float32) + bd_ref[...]
    )
    ps_ref[...] = jnp.dot(h, wr_ref[...], preferred_element_type=jnp.float32)


def _combine_body(x_ref, agg_ref, out_ref):
    out_ref[...] = jnp.maximum(x_ref[...] + agg_ref[0] + agg_ref[1], 0.0)


_row_spec = pl.BlockSpec((ROW_BLK, C), lambda i: (i, 0))
_agg_spec = pl.BlockSpec((NC, ROW_BLK, C), lambda i: (0, i, 0))
_wd_spec = pl.BlockSpec((C, 2 * C), lambda i: (0, 0))
_bd_spec = pl.BlockSpec((1, 2 * C), lambda i: (0, 0))
_p_spec = pl.BlockSpec((ROW_BLK, 2 * C), lambda i: (i, 0))

_proj = pl.pallas_call(
    _proj_body,
    grid=(N // ROW_BLK,),
    in_specs=[_row_spec, _wd_spec, _wd_spec, _bd_spec],
    out_specs=[_p_spec, _p_spec],
    out_shape=[
        jax.ShapeDtypeStruct((NPAD, 2 * C), jnp.float32),
        jax.ShapeDtypeStruct((NPAD, 2 * C), jnp.float32),
    ],
)

_combine_proj = pl.pallas_call(
    _combine_proj_body,
    grid=(N // ROW_BLK,),
    in_specs=[_row_spec, _agg_spec, _wd_spec, _wd_spec, _bd_spec],
    out_specs=[_row_spec, _p_spec, _p_spec],
    out_shape=[
        jax.ShapeDtypeStruct((N, C), jnp.float32),
        jax.ShapeDtypeStruct((NPAD, 2 * C), jnp.float32),
        jax.ShapeDtypeStruct((NPAD, 2 * C), jnp.float32),
    ],
)

_combine = pl.pallas_call(
    _combine_body,
    grid=(N // ROW_BLK,),
    in_specs=[_row_spec, _agg_spec],
    out_specs=_row_spec,
    out_shape=jax.ShapeDtypeStruct((N, C), jnp.float32),
)


def _edge_body(pd_hbm, ps_hbm, idx_hbm, scl_hbm, zero_hbm, out_hbm,
               idx_v, rd_v, rs_v, m_v, scl_v, agg_sh, semd, sems):
    cid = lax.axis_index("c")
    sid = lax.axis_index("s")
    wid = cid * NS + sid
    base = wid * NCHUNK

    pltpu.sync_copy(zero_hbm.at[pl.ds(sid * RPS, RPS)],
                    agg_sh.at[pl.ds(sid * RPS, RPS)])
    pltpu.sync_copy(scl_hbm, scl_v)
    sfv = scl_v[0, pl.ds(0, LANE)]
    ssv = scl_v[1, pl.ds(0, LANE)]
    plsc.subcore_barrier()

    def chunk_body(k, carry):
        pltpu.sync_copy(idx_hbm.at[base + k], idx_v)
        gd = pltpu.async_copy(pd_hbm.at[idx_v.at[0]], rd_v, semd)
        gs = pltpu.async_copy(ps_hbm.at[idx_v.at[1]], rs_v, sems)
        gd.wait()
        gs.wait()

        def row_body(r, c2):
            for j in range(NVEC):
                di = rd_v[r, pl.ds(j * LANE, LANE)]
                si = rs_v[r, pl.ds(j * LANE, LANE)]
                fsum = ((di << 16) >> 16) + ((si << 16) >> 16)
                ssum = (di >> 16) + (si >> 16)
                a = fsum.astype(jnp.float32) * sfv
                b = ssum.astype(jnp.float32) * ssv
                ua = jnp.exp(-jnp.abs(a))
                num = jnp.where(a >= 0.0, 1.0, ua)
                sig = num / (1.0 + ua)
                ub = jnp.exp(-jnp.abs(b))
                w = ub / (ub + 2.0)
                w2 = w * w
                p = w2 * (1.0 / 7.0) + (1.0 / 5.0)
                p = w2 * p + (1.0 / 3.0)
                p = w2 * p + 1.0
                sp = jnp.maximum(b, 0.0) + (2.0 * w) * p
                m_v[r, pl.ds(j * LANE, LANE)] = sig * sp
            return c2

        lax.fori_loop(0, CHUNK, row_body, 0, unroll=2)
        pltpu.sync_copy(m_v, agg_sh.at[idx_v.at[0]], add=True)
        return carry

    lax.fori_loop(0, NCHUNK, chunk_body, 0)
    plsc.subcore_barrier()
    pltpu.sync_copy(agg_sh.at[pl.ds(sid * RPS, RPS)],
                    out_hbm.at[cid, pl.ds(sid * RPS, RPS)])


_edge_phase = functools.partial(
    pl.kernel,
    out_type=jax.ShapeDtypeStruct((NC, NPAD, C), jnp.float32),
    mesh=plsc.VectorSubcoreMesh(core_axis_name="c", subcore_axis_name="s",
                                num_cores=NC, num_subcores=NS),
    scratch_types=[
        pltpu.VMEM((2, CHUNK), jnp.int32),
        pltpu.VMEM((CHUNK, C), jnp.int32),
        pltpu.VMEM((CHUNK, C), jnp.int32),
        pltpu.VMEM((CHUNK, C), jnp.float32),
        pltpu.VMEM((2, C), jnp.float32),
        pltpu.VMEM_SHARED((NPAD, C), jnp.float32),
        pltpu.SemaphoreType.DMA,
        pltpu.SemaphoreType.DMA,
    ],
)(_edge_body)


def _split_weights(Wf, Ws, bf, bs):
    wd = jnp.stack([Wf[:C], Ws[:C]], axis=2).reshape(C, 2 * C)
    wr = jnp.stack([Wf[C:], Ws[C:]], axis=2).reshape(C, 2 * C)
    bd = jnp.stack([bf, bs], axis=1).reshape(1, 2 * C)
    return wd, wr, bd


@jax.jit
def kernel(x, edge_index, W1f, b1f, W1s, b1s, W2f, b2f, W2s, b2s):
    ei = edge_index.astype(jnp.int32)
    pad = jnp.full((NW, EPW_PAD - EPW), NPAD - 1, jnp.int32)
    srcw = jnp.concatenate([ei[0].reshape(NW, EPW), pad], axis=1)
    dstw = jnp.concatenate([ei[1].reshape(NW, EPW), pad], axis=1)
    srcw = srcw.reshape(NW, NCHUNK, CHUNK)
    dstw = dstw.reshape(NW, NCHUNK, CHUNK)
    idx = jnp.stack([dstw, srcw], axis=2).reshape(NW * NCHUNK, 2, CHUNK)
    zero = jnp.zeros((NPAD, C), jnp.float32)

    wd1, wr1, bd1 = _split_weights(W1f, W1s, b1f, b1s)
    wd2, wr2, bd2 = _split_weights(W2f, W2s, b2f, b2s)

    col_is_f = (jnp.arange(2 * C) % 2 == 0)

    def quantize(pd_, ps_):
        af = jnp.maximum(jnp.max(jnp.abs(pd_[:N, 0::2])),
                         jnp.max(jnp.abs(ps_[:N, 0::2])))
        as_ = jnp.maximum(jnp.max(jnp.abs(pd_[:N, 1::2])),
                          jnp.max(jnp.abs(ps_[:N, 1::2])))
        sf = (af + 1e-30) / 32766.0
        ss = (as_ + 1e-30) / 32766.0
        col = jnp.where(col_is_f, sf, ss)

        def topack(p):
            q = jnp.round(p / col).astype(jnp.int16)
            return lax.bitcast_convert_type(q.reshape(NPAD, C, 2), jnp.int32)

        scl = jnp.stack([jnp.broadcast_to(sf, (C,)),
                         jnp.broadcast_to(ss, (C,))]).astype(jnp.float32)
        return topack(pd_), topack(ps_), scl

    pd1, ps1 = _proj(x, wd1, wr1, bd1)
    pq1, sq1, scl1 = quantize(pd1, ps1)
    agg1 = _edge_phase(pq1, sq1, idx, scl1, zero)
    h, pd2, ps2 = _combine_proj(x, agg1, wd2, wr2, bd2)
    pq2, sq2, scl2 = quantize(pd2, ps2)
    agg2 = _edge_phase(pq2, sq2, idx, scl2, zero)
    return _combine(h, agg2)


# speedup vs baseline: 1.5315x; 1.5315x over previous
import functools

import jax
import jax.numpy as jnp
from jax import lax
from jax.experimental import pallas as pl
from jax.experimental.pallas import tpu as pltpu
from jax.experimental.pallas import tpu_sc as plsc

C = 128
N = 10000
E = 320000
NC = 2
NS = 16
NW = NC * NS
EPW = E // NW
CHUNK = 64
NCHUNK = 157
EPW_PAD = NCHUNK * CHUNK
NPAD = 10240
RPS = NPAD // NS
LANE = 16
NVEC = C // LANE

ROW_BLK = 2000


def _proj_body(x_ref, wd_ref, wr_ref, bd_ref, pd_ref, ps_ref):
    xb = x_ref[...]
    pd_ref[...] = (
        jnp.dot(xb, wd_ref[...], preferred_element_type=jnp.float32) + bd_ref[...]
    )
    ps_ref[...] = jnp.dot(xb, wr_ref[...], preferred_element_type=jnp.float32)


def _combine_proj_body(x_ref, agg_ref, wd_ref, wr_ref, bd_ref,
                       h_ref, pd_ref, ps_ref):
    h = jnp.maximum(x_ref[...] + agg_ref[0] + agg_ref[1], 0.0)
    h_ref[...] = h
    pd_ref[...] = (
        jnp.dot(h, wd_ref[...], preferred_element_type=jnp.float32) + bd_ref[...]
    )
    ps_ref[...] = jnp.dot(h, wr_ref[...], preferred_element_type=jnp.float32)


def _combine_body(x_ref, agg_ref, out_ref):
    out_ref[...] = jnp.maximum(x_ref[...] + agg_ref[0] + agg_ref[1], 0.0)


_row_spec = pl.BlockSpec((ROW_BLK, C), lambda i: (i, 0))
_agg_spec = pl.BlockSpec((NC, ROW_BLK, C), lambda i: (0, i, 0))
_wd_spec = pl.BlockSpec((C, 2 * C), lambda i: (0, 0))
_bd_spec = pl.BlockSpec((1, 2 * C), lambda i: (0, 0))
_p_spec = pl.BlockSpec((ROW_BLK, 2 * C), lambda i: (i, 0))

_proj = pl.pallas_call(
    _proj_body,
    grid=(N // ROW_BLK,),
    in_specs=[_row_spec, _wd_spec, _wd_spec, _bd_spec],
    out_specs=[_p_spec, _p_spec],
    out_shape=[
        jax.ShapeDtypeStruct((NPAD, 2 * C), jnp.float32),
        jax.ShapeDtypeStruct((NPAD, 2 * C), jnp.float32),
    ],
)

_combine_proj = pl.pallas_call(
    _combine_proj_body,
    grid=(N // ROW_BLK,),
    in_specs=[_row_spec, _agg_spec, _wd_spec, _wd_spec, _bd_spec],
    out_specs=[_row_spec, _p_spec, _p_spec],
    out_shape=[
        jax.ShapeDtypeStruct((N, C), jnp.float32),
        jax.ShapeDtypeStruct((NPAD, 2 * C), jnp.float32),
        jax.ShapeDtypeStruct((NPAD, 2 * C), jnp.float32),
    ],
)

_combine = pl.pallas_call(
    _combine_body,
    grid=(N // ROW_BLK,),
    in_specs=[_row_spec, _agg_spec],
    out_specs=_row_spec,
    out_shape=jax.ShapeDtypeStruct((N, C), jnp.float32),
)


def _edge_body(pd_hbm, ps_hbm, idx_hbm, zero_hbm, out_hbm,
               idx_v, rd_v, rs_v, m_v, agg_sh, semd, sems):
    cid = lax.axis_index("c")
    sid = lax.axis_index("s")
    wid = cid * NS + sid
    base = wid * NCHUNK

    pltpu.sync_copy(zero_hbm.at[pl.ds(sid * RPS, RPS)],
                    agg_sh.at[pl.ds(sid * RPS, RPS)])
    plsc.subcore_barrier()

    def chunk_body(k, carry):
        pltpu.sync_copy(idx_hbm.at[base + k], idx_v)
        gd = pltpu.async_copy(pd_hbm.at[idx_v.at[0]], rd_v, semd)
        gs = pltpu.async_copy(ps_hbm.at[idx_v.at[1]], rs_v, sems)
        gd.wait()
        gs.wait()

        def row_body(r, c2):
            for j in range(NVEC):
                sl_a = pl.ds(j * LANE, LANE)
                sl_b = pl.ds(C + j * LANE, LANE)
                a = rd_v[r, sl_a] + rs_v[r, sl_a]
                b = rd_v[r, sl_b] + rs_v[r, sl_b]
                ea = jnp.exp(-a)
                ub = jnp.exp(-jnp.abs(b))
                d1 = 1.0 + ea
                d2 = ub + 2.0
                r_ = 1.0 / (d1 * d2)
                sig = r_ * d2
                w = (ub * r_) * d1
                w2 = w * w
                sp = jnp.maximum(b, 0.0) + (2.0 * w) * (w2 * (1.0 / 3.0) + 1.0)
                m_v[r, pl.ds(j * LANE, LANE)] = sig * sp
            return c2

        lax.fori_loop(0, CHUNK, row_body, 0, unroll=4)
        pltpu.sync_copy(m_v, agg_sh.at[idx_v.at[0]], add=True)
        return carry

    lax.fori_loop(0, NCHUNK, chunk_body, 0)
    plsc.subcore_barrier()
    pltpu.sync_copy(agg_sh.at[pl.ds(sid * RPS, RPS)],
                    out_hbm.at[cid, pl.ds(sid * RPS, RPS)])


_edge_phase = functools.partial(
    pl.kernel,
    out_type=jax.ShapeDtypeStruct((NC, NPAD, C), jnp.float32),
    mesh=plsc.VectorSubcoreMesh(core_axis_name="c", subcore_axis_name="s",
                                num_cores=NC, num_subcores=NS),
    scratch_types=[
        pltpu.VMEM((2, CHUNK), jnp.int32),
        pltpu.VMEM((CHUNK, 2 * C), jnp.float32),
        pltpu.VMEM((CHUNK, 2 * C), jnp.float32),
        pltpu.VMEM((CHUNK, C), jnp.float32),
        pltpu.VMEM_SHARED((NPAD, C), jnp.float32),
        pltpu.SemaphoreType.DMA,
        pltpu.SemaphoreType.DMA,
    ],
)(_edge_body)


def _split_weights(Wf, Ws, bf, bs):
    wd = jnp.concatenate([Wf[:C], Ws[:C]], axis=1)
    wr = jnp.concatenate([Wf[C:], Ws[C:]], axis=1)
    bd = jnp.concatenate([bf, bs]).reshape(1, 2 * C)
    return wd, wr, bd


@jax.jit
def kernel(x, edge_index, W1f, b1f, W1s, b1s, W2f, b2f, W2s, b2s):
    ei = edge_index.astype(jnp.int32)
    pad = jnp.full((NW, EPW_PAD - EPW), NPAD - 1, jnp.int32)
    srcw = jnp.concatenate([ei[0].reshape(NW, EPW), pad], axis=1)
    dstw = jnp.concatenate([ei[1].reshape(NW, EPW), pad], axis=1)
    srcw = srcw.reshape(NW, NCHUNK, CHUNK)
    dstw = dstw.reshape(NW, NCHUNK, CHUNK)
    idx = jnp.stack([dstw, srcw], axis=2).reshape(NW * NCHUNK, 2, CHUNK)
    zero = jnp.zeros((NPAD, C), jnp.float32)

    wd1, wr1, bd1 = _split_weights(W1f, W1s, b1f, b1s)
    wd2, wr2, bd2 = _split_weights(W2f, W2s, b2f, b2s)

    pd1, ps1 = _proj(x, wd1, wr1, bd1)
    agg1 = _edge_phase(pd1, ps1, idx, zero)
    h, pd2, ps2 = _combine_proj(x, agg1, wd2, wr2, bd2)
    agg2 = _edge_phase(pd2, ps2, idx, zero)
    return _combine(h, agg2)
